# baseline (speedup 1.0000x reference)
import functools

import jax
import jax.numpy as jnp
from jax import lax
from jax.experimental import pallas as pl
from jax.experimental.pallas import tpu as pltpu
from jax.experimental.pallas import tpu_sc as plsc

N_TYPES = 1000
BATCH = 16384
NC, NS, L = 1, 16, 16
NW = NC * NS
B_PER_W = BATCH // NW


def _log_body(p_ref, o_ref):
    o_ref[...] = jnp.log(p_ref[...])


@functools.partial(
    pl.kernel,
    mesh=plsc.VectorSubcoreMesh(
        core_axis_name="c", subcore_axis_name="s", num_cores=1
    ),
    out_type=jax.ShapeDtypeStruct((BATCH,), jnp.float32),
    scratch_types=[
        pltpu.VMEM((N_TYPES,), jnp.float32),
        pltpu.VMEM((B_PER_W,), jnp.int32),
        pltpu.VMEM((B_PER_W,), jnp.float32),
        pltpu.SemaphoreType.DMA,
        pltpu.SemaphoreType.DMA,
    ],
    compiler_params=pltpu.CompilerParams(needs_layout_passes=False),
)
def _sc_gather(tab_hbm, idx_hbm, out_hbm, tab_v, idx_v, out_v, sem_t, sem_i):
    wid = lax.axis_index("s") * NC + lax.axis_index("c")
    base = wid * B_PER_W
    cp_t = pltpu.async_copy(tab_hbm, tab_v, sem_t)
    cp_i = pltpu.async_copy(idx_hbm.at[pl.ds(base, B_PER_W)], idx_v, sem_i)
    cp_t.wait()
    cp_i.wait()

    def step(i, carry):
        idx = idx_v[pl.ds(i * L, L)]
        out_v[pl.ds(i * L, L)] = plsc.load_gather(tab_v, [idx])
        return carry

    lax.fori_loop(0, B_PER_W // L, step, 0, unroll=8)
    pltpu.sync_copy(out_v, out_hbm.at[pl.ds(base, B_PER_W)])


def kernel(probabilities, c):
    log_tab = pl.pallas_call(
        _log_body,
        out_shape=jax.ShapeDtypeStruct((N_TYPES,), jnp.float32),
    )(probabilities)
    return _sc_gather(log_tab, c.astype(jnp.int32))
